# baseline (speedup 1.0000x reference)
import functools

import jax
import jax.numpy as jnp
from jax import lax
from jax.experimental import pallas as pl
from jax.experimental.pallas import tpu as pltpu
from jax.experimental.pallas import tpu_sc as plsc

S = 10000
NW = 32
NSEGW = (S + NW - 1) // NW
SP = NW * NSEGW
NSP = NSEGW + 1
C = 384
R = 1280


def _mm_stats_kernel(x_ref, wkv_ref, w1_ref, bkv_ref, b1_ref,
                     v_ref, h_ref, st_ref):
    x = x_ref[...]
    v = jnp.dot(x, wkv_ref[...], preferred_element_type=jnp.float32) + bkv_ref[...]
    h = jnp.dot(x, w1_ref[...], preferred_element_type=jnp.float32) + b1_ref[...]
    v_ref[...] = v
    h_ref[...] = h
    st_ref[0] = jnp.concatenate(
        [jnp.sum(v, axis=0)[None], jnp.sum(v * v, axis=0)[None],
         jnp.sum(h, axis=0)[None], jnp.sum(h * h, axis=0)[None]], axis=0)


def _bn_mm_kernel(h_ref, ah_ref, ch_ref, w2_ref, b2_ref, o_ref):
    hx = jnp.maximum(h_ref[...] * ah_ref[...] + ch_ref[...], 0.0)
    o_ref[...] = jnp.dot(hx, w2_ref[...], preferred_element_type=jnp.float32) + b2_ref[...]


def _sc_segment_kernel(v_hbm, w_hbm, ids_hbm, offs_hbm, cst_hbm,
                       wx_hbm, qm_hbm,
                       ids_v, v_v, w_v, qmax, den, num, cnt,
                       offs_v, cst_v):
    f32 = jnp.float32
    i32 = jnp.int32
    cid = lax.axis_index("c")
    sid = lax.axis_index("s")
    wid = (cid * 16 + sid).astype(i32)

    pltpu.sync_copy(offs_hbm, offs_v)
    pltpu.sync_copy(cst_hbm, cst_v)

    win = offs_v[pl.ds(wid, 16)]
    r0 = win[0]
    r1 = win[1]
    s0 = wid * NSEGW
    r0a = (r0 // 8) * 8
    nchunks = (r1 - r0a + (C - 1)) // C

    av = [cst_v[pl.ds(16 * g, 16)] for g in range(4)]
    cv = [cst_v[pl.ds(64 + 16 * g, 16)] for g in range(4)]

    iota = lax.iota(i32, 16)
    trash = jnp.full((16,), NSP - 1, i32)
    minf = jnp.full((16,), -jnp.inf, f32)
    zero = jnp.zeros((16,), f32)
    one = jnp.ones((16,), f32)
    TRASH = NSP - 1

    def init_body(j, _):
        qmax[pl.ds(j * 16, 16)] = minf
        den[pl.ds(j * 16, 16)] = zero
        num[pl.ds(j * 16, 16)] = zero
        return 0
    lax.fori_loop(0, NSP * 64 // 16, init_body, 0)

    def initc_body(j, _):
        cnt[pl.ds(j * 16, 16)] = zero
        return 0
    lax.fori_loop(0, NSP, initc_body, 0)

    def p1_group(g, carry):
        (am0, am1, am2, am3, ad0, ad1, ad2, ad3, an0, an1, an2, an3,
         ax0, ax1, ax2, ax3, ac, prev, row0) = carry
        idvec = ids_v[pl.ds(g * 16, 16)]
        ams = [am0, am1, am2, am3]
        ads = [ad0, ad1, ad2, ad3]
        ans = [an0, an1, an2, an3]
        axs = [ax0, ax1, ax2, ax3]
        rowg = row0 + g * 16
        rowvec = jnp.full((16,), rowg, i32) + iota
        validv = (rowvec >= r0) & (rowvec < r1)
        lidvec = jnp.where(validv, idvec - s0, trash)
        for i in range(16):
            lid_i = lidvec[i]
            bnum = (lid_i != prev).astype(f32)
            prev = lid_i
            bf = jnp.full((16,), bnum, f32)
            nb = one - bf
            off = (g * 16 + i) * 64
            lo = lid_i * 64
            for g2 in range(4):
                wv = w_v[pl.ds(off + 16 * g2, 16)]
                vv = v_v[pl.ds(off + 16 * g2, 16)]
                m_new = bf * wv + nb * jnp.maximum(ams[g2], wv)
                e1 = jnp.exp(jnp.minimum(ams[g2] - m_new, 0.0)) * nb
                e2 = jnp.exp(wv - m_new)
                ams[g2] = m_new
                ads[g2] = ads[g2] * e1 + e2
                den[pl.ds(lo + 16 * g2, 16)] = ads[g2]
                ans[g2] = ans[g2] * e1 + e2 * vv
                num[pl.ds(lo + 16 * g2, 16)] = ans[g2]
                xv = jnp.maximum(vv * av[g2] + cv[g2], 0.0)
                axs[g2] = bf * xv + nb * jnp.maximum(axs[g2], xv)
                qmax[pl.ds(lo + 16 * g2, 16)] = axs[g2]
            ac = ac * nb + one
            cnt[pl.ds(lid_i * 16, 16)] = ac
        return (ams[0], ams[1], ams[2], ams[3], ads[0], ads[1], ads[2], ads[3],
                ans[0], ans[1], ans[2], ans[3], axs[0], axs[1], axs[2], axs[3],
                ac, prev, row0)

    def p1_chunk(c, carry):
        row0 = r0a + c * C
        pltpu.sync_copy(ids_hbm.at[pl.ds(row0, C)], ids_v)
        pltpu.sync_copy(v_hbm.at[pl.ds(row0 * 64, C * 64)], v_v)
        pltpu.sync_copy(w_hbm.at[pl.ds(row0 * 64, C * 64)], w_v)
        st = carry[:18] + (row0,)
        out = lax.fori_loop(0, C // 16, p1_group, st)
        return out[:18]

    init1 = (zero,) * 17 + (jnp.int32(-1),)
    lax.fori_loop(0, nchunks, p1_chunk, init1)

    def epi_body(j, _):
        cg = jnp.maximum(cnt[pl.ds(j * 16, 16)], 1.0)
        for g2 in range(4):
            d = den[pl.ds(j * 64 + 16 * g2, 16)]
            n = num[pl.ds(j * 64 + 16 * g2, 16)]
            num[pl.ds(j * 64 + 16 * g2, 16)] = n / jnp.maximum(d, 1e-30) / cg
        return 0
    lax.fori_loop(0, NSEGW, epi_body, 0)

    pltpu.sync_copy(num.at[pl.ds(0, NSEGW * 64)],
                    wx_hbm.at[pl.ds(s0 * 64, NSEGW * 64)])
    pltpu.sync_copy(qmax.at[pl.ds(0, NSEGW * 64)],
                    qm_hbm.at[pl.ds(s0 * 64, NSEGW * 64)])


def kernel(inputs, unq_inv, W_kv, b_kv, g_norm, be_norm, W1, b1, g_w, be_w, W2, b2):
    f32 = jnp.float32
    i32 = jnp.int32
    N = inputs.shape[0]
    NB = N // R
    N_pad = (NB + 1) * R
    grid = (NB + 1,)

    ids = unq_inv.astype(i32)

    v, h, stats = pl.pallas_call(
        _mm_stats_kernel,
        grid=grid,
        in_specs=[
            pl.BlockSpec((R, 128), lambda i: (jnp.minimum(i, NB - 1), 0)),
            pl.BlockSpec((128, 64), lambda i: (0, 0)),
            pl.BlockSpec((128, 64), lambda i: (0, 0)),
            pl.BlockSpec((1, 64), lambda i: (0, 0)),
            pl.BlockSpec((1, 64), lambda i: (0, 0)),
        ],
        out_specs=[
            pl.BlockSpec((R, 64), lambda i: (i, 0)),
            pl.BlockSpec((R, 64), lambda i: (i, 0)),
            pl.BlockSpec((1, 4, 64), lambda i: (i, 0, 0)),
        ],
        out_shape=[
            jax.ShapeDtypeStruct((N_pad, 64), f32),
            jax.ShapeDtypeStruct((N_pad, 64), f32),
            jax.ShapeDtypeStruct((NB + 1, 4, 64), f32),
        ],
    )(inputs, W_kv, W1, b_kv[None], b1[None])

    sums = jnp.sum(stats[:NB], axis=0)
    mu_v = sums[0] / N
    var_v = sums[1] / N - mu_v * mu_v
    a_v = g_norm * lax.rsqrt(var_v + 1e-3)
    c_v = be_norm - mu_v * a_v
    mu_h = sums[2] / N
    var_h = sums[3] / N - mu_h * mu_h
    a_h = g_w * lax.rsqrt(var_h + 1e-5)
    c_h = be_w - mu_h * a_h

    w = pl.pallas_call(
        _bn_mm_kernel,
        grid=grid,
        in_specs=[
            pl.BlockSpec((R, 64), lambda i: (i, 0)),
            pl.BlockSpec((1, 64), lambda i: (0, 0)),
            pl.BlockSpec((1, 64), lambda i: (0, 0)),
            pl.BlockSpec((64, 64), lambda i: (0, 0)),
            pl.BlockSpec((1, 64), lambda i: (0, 0)),
        ],
        out_specs=pl.BlockSpec((R, 64), lambda i: (i, 0)),
        out_shape=jax.ShapeDtypeStruct((N_pad, 64), f32),
    )(h, a_h[None], c_h[None], W2, b2[None])

    bounds = jnp.arange(NW + 1, dtype=i32) * NSEGW
    offs = jnp.searchsorted(ids, bounds).astype(i32)
    offs = jnp.pad(offs, (0, 48 - (NW + 1)), constant_values=N)
    ids_pad = jnp.pad(ids, (0, N_pad - N), constant_values=S)
    cst = jnp.concatenate([a_v, c_v]).astype(f32)

    mesh = plsc.VectorSubcoreMesh(core_axis_name="c", subcore_axis_name="s")
    sc_fn = functools.partial(
        pl.kernel, mesh=mesh,
        out_type=[
            jax.ShapeDtypeStruct((SP * 64,), f32),
            jax.ShapeDtypeStruct((SP * 64,), f32),
        ],
        scratch_types=[
            pltpu.VMEM((C,), i32),
            pltpu.VMEM((C * 64,), f32),
            pltpu.VMEM((C * 64,), f32),
            pltpu.VMEM((NSP * 64,), f32),
            pltpu.VMEM((NSP * 64,), f32),
            pltpu.VMEM((NSP * 64,), f32),
            pltpu.VMEM((NSP * 16,), f32),
            pltpu.VMEM((48,), i32),
            pltpu.VMEM((128,), f32),
        ],
    )(_sc_segment_kernel)

    wx_flat, qm_flat = sc_fn(
        v.reshape(-1), w.reshape(-1), ids_pad, offs, cst)

    wx = wx_flat.reshape(SP, 64)[:S]
    qm = qm_flat.reshape(SP, 64)[:S]
    return jnp.concatenate([wx, qm], axis=-1)


# speedup vs baseline: 1.2802x; 1.2802x over previous
import functools

import jax
import jax.numpy as jnp
from jax import lax
from jax.experimental import pallas as pl
from jax.experimental.pallas import tpu as pltpu
from jax.experimental.pallas import tpu_sc as plsc

S = 10000
NW = 32
NSEGW = (S + NW - 1) // NW
SP = NW * NSEGW
NSP = NSEGW + 1
C = 384
R = 1280


def _mm_stats_kernel(x_ref, wkv_ref, w1_ref, bkv_ref, b1_ref,
                     v_ref, h_ref, st_ref):
    x = x_ref[...]
    v = jnp.dot(x, wkv_ref[...], preferred_element_type=jnp.float32) + bkv_ref[...]
    h = jnp.dot(x, w1_ref[...], preferred_element_type=jnp.float32) + b1_ref[...]
    v_ref[...] = v
    h_ref[...] = h
    st_ref[0] = jnp.concatenate(
        [jnp.sum(v, axis=0)[None], jnp.sum(v * v, axis=0)[None],
         jnp.sum(h, axis=0)[None], jnp.sum(h * h, axis=0)[None]], axis=0)


def _bn_mm_kernel(h_ref, ah_ref, ch_ref, w2_ref, b2_ref, o_ref):
    hx = jnp.maximum(h_ref[...] * ah_ref[...] + ch_ref[...], 0.0)
    o_ref[...] = jnp.dot(hx, w2_ref[...], preferred_element_type=jnp.float32) + b2_ref[...]


def _sc_segment_kernel(v_hbm, w_hbm, ids_hbm, offs_hbm, cst_hbm,
                       wx_hbm, qm_hbm,
                       ids_v, v_v, w_v, qmax, den, num, cnt,
                       offs_v, cst_v):
    f32 = jnp.float32
    i32 = jnp.int32
    cid = lax.axis_index("c")
    sid = lax.axis_index("s")
    wid = (cid * 16 + sid).astype(i32)

    pltpu.sync_copy(offs_hbm, offs_v)
    pltpu.sync_copy(cst_hbm, cst_v)

    win = offs_v[pl.ds(wid, 16)]
    r0 = win[0]
    r1 = win[1]
    s0 = wid * NSEGW
    r0a = (r0 // 8) * 8
    nchunks = (r1 - r0a + (C - 1)) // C

    av = [cst_v[pl.ds(16 * g, 16)] for g in range(4)]
    cv = [cst_v[pl.ds(64 + 16 * g, 16)] for g in range(4)]

    minf = jnp.full((16,), -jnp.inf, f32)
    zero = jnp.zeros((16,), f32)
    one = jnp.ones((16,), f32)
    TRASH = NSP - 1

    def init_body(j, _):
        qmax[pl.ds(j * 16, 16)] = minf
        den[pl.ds(j * 16, 16)] = zero
        num[pl.ds(j * 16, 16)] = zero
        return 0
    lax.fori_loop(0, NSP * 64 // 16, init_body, 0)

    def initc_body(j, _):
        cnt[pl.ds(j * 16, 16)] = zero
        return 0
    lax.fori_loop(0, NSP, initc_body, 0)

    def p1_group(g, carry):
        (am0, am1, am2, am3, ad0, ad1, ad2, ad3, an0, an1, an2, an3,
         ax0, ax1, ax2, ax3, ac, prev, row0) = carry
        idvec = ids_v[pl.ds(g * 16, 16)]
        ams = [am0, am1, am2, am3]
        ads = [ad0, ad1, ad2, ad3]
        ans = [an0, an1, an2, an3]
        axs = [ax0, ax1, ax2, ax3]
        rowg = row0 + g * 16
        for i in range(16):
            row = rowg + i
            vi = (row >= r0).astype(i32) * (row < r1).astype(i32)
            lid_i = vi * (idvec[i] - s0) + (1 - vi) * TRASH
            bnum = (lid_i != prev).astype(f32)
            prev = lid_i
            bf = jnp.full((16,), bnum, f32)
            nb = one - bf
            off = (g * 16 + i) * 64
            lo = lid_i * 64
            for g2 in range(4):
                wv = w_v[pl.ds(off + 16 * g2, 16)]
                vv = v_v[pl.ds(off + 16 * g2, 16)]
                m_new = bf * wv + nb * jnp.maximum(ams[g2], wv)
                e1 = jnp.exp(jnp.minimum(ams[g2] - m_new, 0.0)) * nb
                e2 = jnp.exp(wv - m_new)
                ams[g2] = m_new
                ads[g2] = ads[g2] * e1 + e2
                den[pl.ds(lo + 16 * g2, 16)] = ads[g2]
                ans[g2] = ans[g2] * e1 + e2 * vv
                num[pl.ds(lo + 16 * g2, 16)] = ans[g2]
                xv = jnp.maximum(vv * av[g2] + cv[g2], 0.0)
                axs[g2] = bf * xv + nb * jnp.maximum(axs[g2], xv)
                qmax[pl.ds(lo + 16 * g2, 16)] = axs[g2]
            ac = ac * nb + one
            cnt[pl.ds(lid_i * 16, 16)] = ac
        return (ams[0], ams[1], ams[2], ams[3], ads[0], ads[1], ads[2], ads[3],
                ans[0], ans[1], ans[2], ans[3], axs[0], axs[1], axs[2], axs[3],
                ac, prev, row0)

    def p1_chunk(c, carry):
        row0 = r0a + c * C
        pltpu.sync_copy(ids_hbm.at[pl.ds(row0, C)], ids_v)
        pltpu.sync_copy(v_hbm.at[pl.ds(row0 * 64, C * 64)], v_v)
        pltpu.sync_copy(w_hbm.at[pl.ds(row0 * 64, C * 64)], w_v)
        st = carry[:18] + (row0,)
        out = lax.fori_loop(0, C // 16, p1_group, st)
        return out[:18]

    init1 = (zero,) * 17 + (jnp.int32(-1),)
    lax.fori_loop(0, nchunks, p1_chunk, init1)

    def epi_body(j, _):
        cg = jnp.maximum(cnt[pl.ds(j * 16, 16)], 1.0)
        for g2 in range(4):
            d = den[pl.ds(j * 64 + 16 * g2, 16)]
            n = num[pl.ds(j * 64 + 16 * g2, 16)]
            num[pl.ds(j * 64 + 16 * g2, 16)] = n / jnp.maximum(d, 1e-30) / cg
        return 0
    lax.fori_loop(0, NSEGW, epi_body, 0)

    pltpu.sync_copy(num.at[pl.ds(0, NSEGW * 64)],
                    wx_hbm.at[pl.ds(s0 * 64, NSEGW * 64)])
    pltpu.sync_copy(qmax.at[pl.ds(0, NSEGW * 64)],
                    qm_hbm.at[pl.ds(s0 * 64, NSEGW * 64)])


def kernel(inputs, unq_inv, W_kv, b_kv, g_norm, be_norm, W1, b1, g_w, be_w, W2, b2):
    f32 = jnp.float32
    i32 = jnp.int32
    N = inputs.shape[0]
    NB = N // R
    N_pad = (NB + 1) * R
    grid = (NB + 1,)

    ids = unq_inv.astype(i32)

    v, h, stats = pl.pallas_call(
        _mm_stats_kernel,
        grid=grid,
        in_specs=[
            pl.BlockSpec((R, 128), lambda i: (jnp.minimum(i, NB - 1), 0)),
            pl.BlockSpec((128, 64), lambda i: (0, 0)),
            pl.BlockSpec((128, 64), lambda i: (0, 0)),
            pl.BlockSpec((1, 64), lambda i: (0, 0)),
            pl.BlockSpec((1, 64), lambda i: (0, 0)),
        ],
        out_specs=[
            pl.BlockSpec((R, 64), lambda i: (i, 0)),
            pl.BlockSpec((R, 64), lambda i: (i, 0)),
            pl.BlockSpec((1, 4, 64), lambda i: (i, 0, 0)),
        ],
        out_shape=[
            jax.ShapeDtypeStruct((N_pad, 64), f32),
            jax.ShapeDtypeStruct((N_pad, 64), f32),
            jax.ShapeDtypeStruct((NB + 1, 4, 64), f32),
        ],
    )(inputs, W_kv, W1, b_kv[None], b1[None])

    sums = jnp.sum(stats[:NB], axis=0)
    mu_v = sums[0] / N
    var_v = sums[1] / N - mu_v * mu_v
    a_v = g_norm * lax.rsqrt(var_v + 1e-3)
    c_v = be_norm - mu_v * a_v
    mu_h = sums[2] / N
    var_h = sums[3] / N - mu_h * mu_h
    a_h = g_w * lax.rsqrt(var_h + 1e-5)
    c_h = be_w - mu_h * a_h

    w = pl.pallas_call(
        _bn_mm_kernel,
        grid=grid,
        in_specs=[
            pl.BlockSpec((R, 64), lambda i: (i, 0)),
            pl.BlockSpec((1, 64), lambda i: (0, 0)),
            pl.BlockSpec((1, 64), lambda i: (0, 0)),
            pl.BlockSpec((64, 64), lambda i: (0, 0)),
            pl.BlockSpec((1, 64), lambda i: (0, 0)),
        ],
        out_specs=pl.BlockSpec((R, 64), lambda i: (i, 0)),
        out_shape=jax.ShapeDtypeStruct((N_pad, 64), f32),
    )(h, a_h[None], c_h[None], W2, b2[None])

    bounds = jnp.arange(NW + 1, dtype=i32) * NSEGW
    offs = jnp.searchsorted(ids, bounds).astype(i32)
    offs = jnp.pad(offs, (0, 48 - (NW + 1)), constant_values=N)
    ids_pad = jnp.pad(ids, (0, N_pad - N), constant_values=S)
    cst = jnp.concatenate([a_v, c_v]).astype(f32)

    mesh = plsc.VectorSubcoreMesh(core_axis_name="c", subcore_axis_name="s")
    sc_fn = functools.partial(
        pl.kernel, mesh=mesh,
        out_type=[
            jax.ShapeDtypeStruct((SP * 64,), f32),
            jax.ShapeDtypeStruct((SP * 64,), f32),
        ],
        scratch_types=[
            pltpu.VMEM((C,), i32),
            pltpu.VMEM((C * 64,), f32),
            pltpu.VMEM((C * 64,), f32),
            pltpu.VMEM((NSP * 64,), f32),
            pltpu.VMEM((NSP * 64,), f32),
            pltpu.VMEM((NSP * 64,), f32),
            pltpu.VMEM((NSP * 16,), f32),
            pltpu.VMEM((48,), i32),
            pltpu.VMEM((128,), f32),
        ],
    )(_sc_segment_kernel)

    wx_flat, qm_flat = sc_fn(
        v.reshape(-1), w.reshape(-1), ids_pad, offs, cst)

    wx = wx_flat.reshape(SP, 64)[:S]
    qm = qm_flat.reshape(SP, 64)[:S]
    return jnp.concatenate([wx, qm], axis=-1)


# speedup vs baseline: 1.2809x; 1.0005x over previous
import functools

import jax
import jax.numpy as jnp
from jax import lax
from jax.experimental import pallas as pl
from jax.experimental.pallas import tpu as pltpu
from jax.experimental.pallas import tpu_sc as plsc

S = 10000
NW = 32
NSEGW = (S + NW - 1) // NW
SP = NW * NSEGW
NSP = NSEGW + 1
C = 384
R = 1280


def _mm_stats_kernel(x_ref, wkv_ref, w1_ref, bkv_ref, b1_ref,
                     v_ref, h_ref, st_ref):
    x = x_ref[...]
    v = jnp.dot(x, wkv_ref[...], preferred_element_type=jnp.float32) + bkv_ref[...]
    h = jnp.dot(x, w1_ref[...], preferred_element_type=jnp.float32) + b1_ref[...]
    v_ref[...] = v
    h_ref[...] = h
    st_ref[0] = jnp.concatenate(
        [jnp.sum(v, axis=0)[None], jnp.sum(v * v, axis=0)[None],
         jnp.sum(h, axis=0)[None], jnp.sum(h * h, axis=0)[None]], axis=0)


def _bn_mm_kernel(h_ref, ah_ref, ch_ref, w2_ref, b2_ref, o_ref):
    hx = jnp.maximum(h_ref[...] * ah_ref[...] + ch_ref[...], 0.0)
    o_ref[...] = jnp.dot(hx, w2_ref[...], preferred_element_type=jnp.float32) + b2_ref[...]


def _sc_segment_kernel(v_hbm, w_hbm, ids_hbm, offs_hbm, cst_hbm,
                       wx_hbm, qm_hbm,
                       ids_v, v_v, w_v, qmax, den, num, cnt,
                       offs_v, cst_v):
    f32 = jnp.float32
    i32 = jnp.int32
    cid = lax.axis_index("c")
    sid = lax.axis_index("s")
    wid = (cid * 16 + sid).astype(i32)

    pltpu.sync_copy(offs_hbm, offs_v)
    pltpu.sync_copy(cst_hbm, cst_v)

    win = offs_v[pl.ds(wid, 16)]
    r0 = win[0]
    r1 = win[1]
    s0 = wid * NSEGW
    r0a = (r0 // 8) * 8
    nchunks = (r1 - r0a + (C - 1)) // C

    av = [cst_v[pl.ds(16 * g, 16)] for g in range(4)]
    cv = [cst_v[pl.ds(64 + 16 * g, 16)] for g in range(4)]

    minf = jnp.full((16,), -jnp.inf, f32)
    zero = jnp.zeros((16,), f32)
    one = jnp.ones((16,), f32)
    TRASH = NSP - 1

    def init_body(j, _):
        qmax[pl.ds(j * 16, 16)] = minf
        den[pl.ds(j * 16, 16)] = zero
        num[pl.ds(j * 16, 16)] = zero
        return 0
    lax.fori_loop(0, NSP * 64 // 16, init_body, 0)

    def initc_body(j, _):
        cnt[pl.ds(j * 16, 16)] = zero
        return 0
    lax.fori_loop(0, NSP, initc_body, 0)

    def p1_group(g, carry, guarded=True):
        (am0, am1, am2, am3, ad0, ad1, ad2, ad3, an0, an1, an2, an3,
         ax0, ax1, ax2, ax3, ac, prev, row0) = carry
        idvec = ids_v[pl.ds(g * 16, 16)]
        ams = [am0, am1, am2, am3]
        ads = [ad0, ad1, ad2, ad3]
        ans = [an0, an1, an2, an3]
        axs = [ax0, ax1, ax2, ax3]
        rowg = row0 + g * 16
        for i in range(16):
            if guarded:
                row = rowg + i
                vi = (row >= r0).astype(i32) * (row < r1).astype(i32)
                lid_i = vi * (idvec[i] - s0) + (1 - vi) * TRASH
            else:
                lid_i = idvec[i] - s0
            bnum = (lid_i != prev).astype(f32)
            prev = lid_i
            bf = jnp.full((16,), bnum, f32)
            nb = one - bf
            off = (g * 16 + i) * 64
            lo = lid_i * 64
            for g2 in range(4):
                wv = w_v[pl.ds(off + 16 * g2, 16)]
                vv = v_v[pl.ds(off + 16 * g2, 16)]
                m_new = bf * wv + nb * jnp.maximum(ams[g2], wv)
                e1 = jnp.exp(jnp.minimum(ams[g2] - m_new, 0.0)) * nb
                e2 = jnp.exp(wv - m_new)
                ams[g2] = m_new
                ads[g2] = ads[g2] * e1 + e2
                den[pl.ds(lo + 16 * g2, 16)] = ads[g2]
                ans[g2] = ans[g2] * e1 + e2 * vv
                num[pl.ds(lo + 16 * g2, 16)] = ans[g2]
                xv = jnp.maximum(vv * av[g2] + cv[g2], 0.0)
                axs[g2] = bf * xv + nb * jnp.maximum(axs[g2], xv)
                qmax[pl.ds(lo + 16 * g2, 16)] = axs[g2]
            ac = ac * nb + one
            cnt[pl.ds(lid_i * 16, 16)] = ac
        return (ams[0], ams[1], ams[2], ams[3], ads[0], ads[1], ads[2], ads[3],
                ans[0], ans[1], ans[2], ans[3], axs[0], axs[1], axs[2], axs[3],
                ac, prev, row0)

    def p1_chunk(c, carry, guarded=True):
        row0 = r0a + c * C
        pltpu.sync_copy(ids_hbm.at[pl.ds(row0, C)], ids_v)
        pltpu.sync_copy(v_hbm.at[pl.ds(row0 * 64, C * 64)], v_v)
        pltpu.sync_copy(w_hbm.at[pl.ds(row0 * 64, C * 64)], w_v)
        st = carry[:18] + (row0,)
        body = p1_group if guarded else functools.partial(p1_group, guarded=False)
        out = lax.fori_loop(0, C // 16, body, st)
        return out[:18]

    init1 = (zero,) * 17 + (jnp.int32(-1),)
    st = p1_chunk(0, init1 + (jnp.int32(0),))
    st = lax.fori_loop(1, jnp.maximum(nchunks - 1, 1),
                       functools.partial(p1_chunk, guarded=False), st)
    st = lax.fori_loop(jnp.maximum(nchunks - 1, 1), nchunks, p1_chunk, st)

    def epi_body(j, _):
        cg = jnp.maximum(cnt[pl.ds(j * 16, 16)], 1.0)
        for g2 in range(4):
            d = den[pl.ds(j * 64 + 16 * g2, 16)]
            n = num[pl.ds(j * 64 + 16 * g2, 16)]
            num[pl.ds(j * 64 + 16 * g2, 16)] = n / jnp.maximum(d, 1e-30) / cg
        return 0
    lax.fori_loop(0, NSEGW, epi_body, 0)

    pltpu.sync_copy(num.at[pl.ds(0, NSEGW * 64)],
                    wx_hbm.at[pl.ds(s0 * 64, NSEGW * 64)])
    pltpu.sync_copy(qmax.at[pl.ds(0, NSEGW * 64)],
                    qm_hbm.at[pl.ds(s0 * 64, NSEGW * 64)])


def kernel(inputs, unq_inv, W_kv, b_kv, g_norm, be_norm, W1, b1, g_w, be_w, W2, b2):
    f32 = jnp.float32
    i32 = jnp.int32
    N = inputs.shape[0]
    NB = N // R
    N_pad = (NB + 1) * R
    grid = (NB + 1,)

    ids = unq_inv.astype(i32)

    v, h, stats = pl.pallas_call(
        _mm_stats_kernel,
        grid=grid,
        in_specs=[
            pl.BlockSpec((R, 128), lambda i: (jnp.minimum(i, NB - 1), 0)),
            pl.BlockSpec((128, 64), lambda i: (0, 0)),
            pl.BlockSpec((128, 64), lambda i: (0, 0)),
            pl.BlockSpec((1, 64), lambda i: (0, 0)),
            pl.BlockSpec((1, 64), lambda i: (0, 0)),
        ],
        out_specs=[
            pl.BlockSpec((R, 64), lambda i: (i, 0)),
            pl.BlockSpec((R, 64), lambda i: (i, 0)),
            pl.BlockSpec((1, 4, 64), lambda i: (i, 0, 0)),
        ],
        out_shape=[
            jax.ShapeDtypeStruct((N_pad, 64), f32),
            jax.ShapeDtypeStruct((N_pad, 64), f32),
            jax.ShapeDtypeStruct((NB + 1, 4, 64), f32),
        ],
    )(inputs, W_kv, W1, b_kv[None], b1[None])

    sums = jnp.sum(stats[:NB], axis=0)
    mu_v = sums[0] / N
    var_v = sums[1] / N - mu_v * mu_v
    a_v = g_norm * lax.rsqrt(var_v + 1e-3)
    c_v = be_norm - mu_v * a_v
    mu_h = sums[2] / N
    var_h = sums[3] / N - mu_h * mu_h
    a_h = g_w * lax.rsqrt(var_h + 1e-5)
    c_h = be_w - mu_h * a_h

    w = pl.pallas_call(
        _bn_mm_kernel,
        grid=grid,
        in_specs=[
            pl.BlockSpec((R, 64), lambda i: (i, 0)),
            pl.BlockSpec((1, 64), lambda i: (0, 0)),
            pl.BlockSpec((1, 64), lambda i: (0, 0)),
            pl.BlockSpec((64, 64), lambda i: (0, 0)),
            pl.BlockSpec((1, 64), lambda i: (0, 0)),
        ],
        out_specs=pl.BlockSpec((R, 64), lambda i: (i, 0)),
        out_shape=jax.ShapeDtypeStruct((N_pad, 64), f32),
    )(h, a_h[None], c_h[None], W2, b2[None])

    bounds = jnp.arange(NW + 1, dtype=i32) * NSEGW
    offs = jnp.searchsorted(ids, bounds).astype(i32)
    offs = jnp.pad(offs, (0, 48 - (NW + 1)), constant_values=N)
    ids_pad = jnp.pad(ids, (0, N_pad - N), constant_values=S)
    cst = jnp.concatenate([a_v, c_v]).astype(f32)

    mesh = plsc.VectorSubcoreMesh(core_axis_name="c", subcore_axis_name="s")
    sc_fn = functools.partial(
        pl.kernel, mesh=mesh,
        out_type=[
            jax.ShapeDtypeStruct((SP * 64,), f32),
            jax.ShapeDtypeStruct((SP * 64,), f32),
        ],
        scratch_types=[
            pltpu.VMEM((C,), i32),
            pltpu.VMEM((C * 64,), f32),
            pltpu.VMEM((C * 64,), f32),
            pltpu.VMEM((NSP * 64,), f32),
            pltpu.VMEM((NSP * 64,), f32),
            pltpu.VMEM((NSP * 64,), f32),
            pltpu.VMEM((NSP * 16,), f32),
            pltpu.VMEM((48,), i32),
            pltpu.VMEM((128,), f32),
        ],
    )(_sc_segment_kernel)

    wx_flat, qm_flat = sc_fn(
        v.reshape(-1), w.reshape(-1), ids_pad, offs, cst)

    wx = wx_flat.reshape(SP, 64)[:S]
    qm = qm_flat.reshape(SP, 64)[:S]
    return jnp.concatenate([wx, qm], axis=-1)


# speedup vs baseline: 1.2964x; 1.0122x over previous
import functools

import jax
import jax.numpy as jnp
from jax import lax
from jax.experimental import pallas as pl
from jax.experimental.pallas import tpu as pltpu
from jax.experimental.pallas import tpu_sc as plsc

S = 10000
NW = 32
NSEGW = (S + NW - 1) // NW
SP = NW * NSEGW
NSP = NSEGW + 1
C = 192
R = 1280


def _mm_stats_kernel(x_ref, wkv_ref, w1_ref, bkv_ref, b1_ref,
                     v_ref, h_ref, st_ref):
    x = x_ref[...]
    v = jnp.dot(x, wkv_ref[...], preferred_element_type=jnp.float32) + bkv_ref[...]
    h = jnp.dot(x, w1_ref[...], preferred_element_type=jnp.float32) + b1_ref[...]
    v_ref[...] = v
    h_ref[...] = h
    st_ref[0] = jnp.concatenate(
        [jnp.sum(v, axis=0)[None], jnp.sum(v * v, axis=0)[None],
         jnp.sum(h, axis=0)[None], jnp.sum(h * h, axis=0)[None]], axis=0)


def _bn_mm_kernel(h_ref, ah_ref, ch_ref, w2_ref, b2_ref, o_ref):
    hx = jnp.maximum(h_ref[...] * ah_ref[...] + ch_ref[...], 0.0)
    o_ref[...] = jnp.dot(hx, w2_ref[...], preferred_element_type=jnp.float32) + b2_ref[...]


def _sc_segment_kernel(v_hbm, w_hbm, ids_hbm, offs_hbm, cst_hbm,
                       wx_hbm, qm_hbm,
                       ids_v, v_v, w_v, ids_v1, v_v1, w_v1, qmax, den, num,
                       cnt, offs_v, cst_v, sem0, sem1):
    f32 = jnp.float32
    i32 = jnp.int32
    cid = lax.axis_index("c")
    sid = lax.axis_index("s")
    wid = (cid * 16 + sid).astype(i32)

    pltpu.sync_copy(offs_hbm, offs_v)
    pltpu.sync_copy(cst_hbm, cst_v)

    win = offs_v[pl.ds(wid, 16)]
    r0 = win[0]
    r1 = win[1]
    s0 = wid * NSEGW
    r0a = (r0 // 8) * 8
    nchunks = (r1 - r0a + (C - 1)) // C

    av = [cst_v[pl.ds(16 * g, 16)] for g in range(4)]
    cv = [cst_v[pl.ds(64 + 16 * g, 16)] for g in range(4)]

    minf = jnp.full((16,), -jnp.inf, f32)
    zero = jnp.zeros((16,), f32)
    one = jnp.ones((16,), f32)
    TRASH = NSP - 1

    def init_body(j, _):
        qmax[pl.ds(j * 16, 16)] = minf
        den[pl.ds(j * 16, 16)] = zero
        num[pl.ds(j * 16, 16)] = zero
        return 0
    lax.fori_loop(0, NSP * 64 // 16, init_body, 0)

    def initc_body(j, _):
        cnt[pl.ds(j * 16, 16)] = zero
        return 0
    lax.fori_loop(0, NSP, initc_body, 0)

    def p1_group(g, carry, bufs):
        idsb, vb, wb = bufs
        (am0, am1, am2, am3, ad0, ad1, ad2, ad3, an0, an1, an2, an3,
         ax0, ax1, ax2, ax3, ac, prev, row0) = carry
        idvec = idsb[pl.ds(g * 16, 16)]
        ams = [am0, am1, am2, am3]
        ads = [ad0, ad1, ad2, ad3]
        ans = [an0, an1, an2, an3]
        axs = [ax0, ax1, ax2, ax3]
        rowg = row0 + g * 16
        for i in range(16):
            row = rowg + i
            vi = (row >= r0).astype(i32) * (row < r1).astype(i32)
            lid_i = vi * (idvec[i] - s0) + (1 - vi) * TRASH
            bnum = (lid_i != prev).astype(f32)
            prev = lid_i
            bf = jnp.full((16,), bnum, f32)
            nb = one - bf
            off = (g * 16 + i) * 64
            lo = lid_i * 64
            for g2 in range(4):
                wv = wb[pl.ds(off + 16 * g2, 16)]
                vv = vb[pl.ds(off + 16 * g2, 16)]
                m_new = bf * wv + nb * jnp.maximum(ams[g2], wv)
                e1 = jnp.exp(jnp.minimum(ams[g2] - m_new, 0.0)) * nb
                e2 = jnp.exp(wv - m_new)
                ams[g2] = m_new
                ads[g2] = ads[g2] * e1 + e2
                den[pl.ds(lo + 16 * g2, 16)] = ads[g2]
                ans[g2] = ans[g2] * e1 + e2 * vv
                num[pl.ds(lo + 16 * g2, 16)] = ans[g2]
                xv = jnp.maximum(vv * av[g2] + cv[g2], 0.0)
                axs[g2] = bf * xv + nb * jnp.maximum(axs[g2], xv)
                qmax[pl.ds(lo + 16 * g2, 16)] = axs[g2]
            ac = ac * nb + one
            cnt[pl.ds(lid_i * 16, 16)] = ac
        return (ams[0], ams[1], ams[2], ams[3], ads[0], ads[1], ads[2], ads[3],
                ans[0], ans[1], ans[2], ans[3], axs[0], axs[1], axs[2], axs[3],
                ac, prev, row0)

    B0 = (ids_v, v_v, w_v, sem0)
    B1 = (ids_v1, v_v1, w_v1, sem1)

    def _dmas(c, buf):
        idsb, vb, wb, sem = buf
        row0 = r0a + c * C
        return (pltpu.make_async_copy(ids_hbm.at[pl.ds(row0, C)], idsb, sem),
                pltpu.make_async_copy(v_hbm.at[pl.ds(row0 * 64, C * 64)], vb, sem),
                pltpu.make_async_copy(w_hbm.at[pl.ds(row0 * 64, C * 64)], wb, sem))

    def start_chunk(c, buf):
        for d in _dmas(c, buf):
            d.start()

    def wait_chunk(c, buf):
        for d in _dmas(c, buf):
            d.wait()

    def compute_chunk(c, carry, buf):
        st = carry[:18] + (r0a + c * C,)
        out = lax.fori_loop(0, C // 16,
                            functools.partial(p1_group, bufs=buf[:3]), st)
        return out[:18]

    def pair(k, carry):
        c0 = 2 * k
        wait_chunk(c0, B0)
        start_chunk(c0 + 1, B1)
        carry = compute_chunk(c0, carry, B0)
        wait_chunk(c0 + 1, B1)
        start_chunk(c0 + 2, B0)
        carry = compute_chunk(c0 + 1, carry, B1)
        return carry

    nch2 = (nchunks + 1) // 2
    init1 = (zero,) * 17 + (jnp.int32(-1),)
    start_chunk(0, B0)
    lax.fori_loop(0, nch2, pair, init1)
    wait_chunk(2 * nch2, B0)

    def epi_body(j, _):
        cg = jnp.maximum(cnt[pl.ds(j * 16, 16)], 1.0)
        for g2 in range(4):
            d = den[pl.ds(j * 64 + 16 * g2, 16)]
            n = num[pl.ds(j * 64 + 16 * g2, 16)]
            num[pl.ds(j * 64 + 16 * g2, 16)] = n / jnp.maximum(d, 1e-30) / cg
        return 0
    lax.fori_loop(0, NSEGW, epi_body, 0)

    pltpu.sync_copy(num.at[pl.ds(0, NSEGW * 64)],
                    wx_hbm.at[pl.ds(s0 * 64, NSEGW * 64)])
    pltpu.sync_copy(qmax.at[pl.ds(0, NSEGW * 64)],
                    qm_hbm.at[pl.ds(s0 * 64, NSEGW * 64)])


def kernel(inputs, unq_inv, W_kv, b_kv, g_norm, be_norm, W1, b1, g_w, be_w, W2, b2):
    f32 = jnp.float32
    i32 = jnp.int32
    N = inputs.shape[0]
    NB = N // R
    N_pad = (NB + 1) * R
    grid = (NB + 1,)

    ids = unq_inv.astype(i32)

    v, h, stats = pl.pallas_call(
        _mm_stats_kernel,
        grid=grid,
        in_specs=[
            pl.BlockSpec((R, 128), lambda i: (jnp.minimum(i, NB - 1), 0)),
            pl.BlockSpec((128, 64), lambda i: (0, 0)),
            pl.BlockSpec((128, 64), lambda i: (0, 0)),
            pl.BlockSpec((1, 64), lambda i: (0, 0)),
            pl.BlockSpec((1, 64), lambda i: (0, 0)),
        ],
        out_specs=[
            pl.BlockSpec((R, 64), lambda i: (i, 0)),
            pl.BlockSpec((R, 64), lambda i: (i, 0)),
            pl.BlockSpec((1, 4, 64), lambda i: (i, 0, 0)),
        ],
        out_shape=[
            jax.ShapeDtypeStruct((N_pad, 64), f32),
            jax.ShapeDtypeStruct((N_pad, 64), f32),
            jax.ShapeDtypeStruct((NB + 1, 4, 64), f32),
        ],
    )(inputs, W_kv, W1, b_kv[None], b1[None])

    sums = jnp.sum(stats[:NB], axis=0)
    mu_v = sums[0] / N
    var_v = sums[1] / N - mu_v * mu_v
    a_v = g_norm * lax.rsqrt(var_v + 1e-3)
    c_v = be_norm - mu_v * a_v
    mu_h = sums[2] / N
    var_h = sums[3] / N - mu_h * mu_h
    a_h = g_w * lax.rsqrt(var_h + 1e-5)
    c_h = be_w - mu_h * a_h

    w = pl.pallas_call(
        _bn_mm_kernel,
        grid=grid,
        in_specs=[
            pl.BlockSpec((R, 64), lambda i: (i, 0)),
            pl.BlockSpec((1, 64), lambda i: (0, 0)),
            pl.BlockSpec((1, 64), lambda i: (0, 0)),
            pl.BlockSpec((64, 64), lambda i: (0, 0)),
            pl.BlockSpec((1, 64), lambda i: (0, 0)),
        ],
        out_specs=pl.BlockSpec((R, 64), lambda i: (i, 0)),
        out_shape=jax.ShapeDtypeStruct((N_pad, 64), f32),
    )(h, a_h[None], c_h[None], W2, b2[None])

    bounds = jnp.arange(NW + 1, dtype=i32) * NSEGW
    offs = jnp.searchsorted(ids, bounds).astype(i32)
    offs = jnp.pad(offs, (0, 48 - (NW + 1)), constant_values=N)
    ids_pad = jnp.pad(ids, (0, N_pad - N), constant_values=S)
    cst = jnp.concatenate([a_v, c_v]).astype(f32)

    mesh = plsc.VectorSubcoreMesh(core_axis_name="c", subcore_axis_name="s")
    sc_fn = functools.partial(
        pl.kernel, mesh=mesh,
        out_type=[
            jax.ShapeDtypeStruct((SP * 64,), f32),
            jax.ShapeDtypeStruct((SP * 64,), f32),
        ],
        scratch_types=[
            pltpu.VMEM((C,), i32),
            pltpu.VMEM((C * 64,), f32),
            pltpu.VMEM((C * 64,), f32),
            pltpu.VMEM((C,), i32),
            pltpu.VMEM((C * 64,), f32),
            pltpu.VMEM((C * 64,), f32),
            pltpu.VMEM((NSP * 64,), f32),
            pltpu.VMEM((NSP * 64,), f32),
            pltpu.VMEM((NSP * 64,), f32),
            pltpu.VMEM((NSP * 16,), f32),
            pltpu.VMEM((48,), i32),
            pltpu.VMEM((128,), f32),
            pltpu.SemaphoreType.DMA,
            pltpu.SemaphoreType.DMA,
        ],
    )(_sc_segment_kernel)

    wx_flat, qm_flat = sc_fn(
        v.reshape(-1), w.reshape(-1), ids_pad, offs, cst)

    wx = wx_flat.reshape(SP, 64)[:S]
    qm = qm_flat.reshape(SP, 64)[:S]
    return jnp.concatenate([wx, qm], axis=-1)
